# baseline (speedup 1.0000x reference)
import functools

import jax
import jax.numpy as jnp
from jax import lax
from jax.experimental import pallas as pl
from jax.experimental.pallas import tpu as pltpu
from jax.experimental.pallas import tpu_sc as plsc

_N = 10000
_C = 128
_H = 32
_K = 43
_EPS = 1e-5


def _fc1_body(feat_ref, w_ref, g_ref, b_ref, out_ref):
    f = feat_ref[...]
    w = w_ref[...]
    xw = lax.dot_general(f, w, (((1,), (1,)), ((), ())),
                         preferred_element_type=jnp.float32)
    s = jnp.sum(f, axis=0, keepdims=True)
    mean = lax.dot_general(s, w, (((1,), (1,)), ((), ()))) / _N
    gram = lax.dot_general(f, f, (((0,), (0,)), ((), ())),
                           preferred_element_type=jnp.float32)
    m = lax.dot_general(w, gram, (((1,), (0,)), ((), ())),
                        preferred_element_type=jnp.float32)
    q = (jnp.sum(m * w, axis=1) / _N)[None, :]
    var = q - mean * mean
    scale = g_ref[...] * lax.rsqrt(var + _EPS)
    shift = b_ref[...] - mean * scale
    out_ref[...] = jnp.maximum(xw * scale + shift, 0.0)


def _fc1(feat, w1, g1, b1):
    return pl.pallas_call(
        _fc1_body,
        out_shape=jax.ShapeDtypeStruct((_N, _C), jnp.float32),
    )(feat, w1, g1, b1)


_NW = 32
_ECH = 128
_NSL = 5
_PSL = _N // _NSL
_ESL = _PSL * _H
_NCHS = _ESL // _ECH
_NCHW = _NCHS // _NW
_LEFT = _NCHS - _NCHW * _NW


def _sc_gather_body(xt_ref, ct_ref, idx_ref, nf_ref, np_ref,
                    idx0, idx1, rows0, rows1, crows0, crows1,
                    semf0, semf1, semc0, semc1):
    cid = lax.axis_index("c")
    sid = lax.axis_index("s")
    wid = sid * 2 + cid

    def fire(j, idx_v, rows_v, crows_v, semf, semc):
        base = (j * _NW + wid) * _ECH
        pltpu.sync_copy(idx_ref.at[pl.ds(base, _ECH)], idx_v)
        pltpu.async_copy(xt_ref.at[idx_v], rows_v, semf)
        pltpu.async_copy(ct_ref.at[idx_v], crows_v, semc)

    def drain(j, idx_v, rows_v, crows_v, semf, semc):
        base = (j * _NW + wid) * _ECH
        pltpu.make_async_copy(xt_ref.at[idx_v], rows_v, semf).wait()
        pltpu.make_async_copy(ct_ref.at[idx_v], crows_v, semc).wait()
        pltpu.sync_copy(rows_v, nf_ref.at[pl.ds(base, _ECH)])
        pltpu.sync_copy(crows_v, np_ref.at[pl.ds(base, _ECH)])

    fire(0, idx0, rows0, crows0, semf0, semc0)

    def step(jp, carry):
        j1 = 2 * jp + 1
        j2 = 2 * jp + 2

        @pl.when(j1 < _NCHW)
        def _():
            fire(j1, idx1, rows1, crows1, semf1, semc1)

        drain(2 * jp, idx0, rows0, crows0, semf0, semc0)

        @pl.when(j2 < _NCHW)
        def _():
            fire(j2, idx0, rows0, crows0, semf0, semc0)

        @pl.when(j1 < _NCHW)
        def _():
            drain(j1, idx1, rows1, crows1, semf1, semc1)

        return carry

    lax.fori_loop(0, (_NCHW + 1) // 2, step, 0)

    @pl.when(wid < _LEFT)
    def _():
        fire(_NCHW, idx0, rows0, crows0, semf0, semc0)
        drain(_NCHW, idx0, rows0, crows0, semf0, semc0)


def _sc_gather(x1, coordpad, idx_slice):
    k = pl.kernel(
        _sc_gather_body,
        out_type=[jax.ShapeDtypeStruct((_ESL, _C), jnp.float32),
                  jax.ShapeDtypeStruct((_ESL, 16), jnp.float32)],
        mesh=plsc.VectorSubcoreMesh(core_axis_name="c", subcore_axis_name="s"),
        scratch_types=[pltpu.VMEM((_ECH,), jnp.int32),
                       pltpu.VMEM((_ECH,), jnp.int32),
                       pltpu.VMEM((_ECH, _C), jnp.float32),
                       pltpu.VMEM((_ECH, _C), jnp.float32),
                       pltpu.VMEM((_ECH, 16), jnp.float32),
                       pltpu.VMEM((_ECH, 16), jnp.float32),
                       pltpu.SemaphoreType.DMA,
                       pltpu.SemaphoreType.DMA,
                       pltpu.SemaphoreType.DMA,
                       pltpu.SemaphoreType.DMA],
        compiler_params=pltpu.CompilerParams(use_tc_tiling_on_sc=False),
    )
    return k(x1, coordpad, idx_slice)


_P = 200
_E = _P * _H
_NB = _N // _P


def _conv_body(nf_ref, np_ref, cp_ref, kpt_ref, w_ref, out_ref):
    npos = np_ref[...]
    ctr = cp_ref[...]
    rel = (npos.reshape(_P, _H, 16) - ctr.reshape(_P, 1, 16)).reshape(_E, 16)
    kpt = kpt_ref[...]
    rn2 = jnp.sum(rel * rel, axis=1, keepdims=True)
    dotp = lax.dot_general(rel, kpt, (((1,), (0,)), ((), ())),
                           precision=lax.Precision.HIGHEST,
                           preferred_element_type=jnp.float32)
    kn2 = jnp.sum(kpt * kpt, axis=0, keepdims=True)
    sq = jnp.maximum(rn2 - 2.0 * dotp + kn2, 0.0)
    infl = jnp.maximum(1.0 - jnp.sqrt(sq), 0.0)
    coeff = lax.dot_general(infl.astype(jnp.bfloat16), w_ref[...],
                            (((1,), (0,)), ((), ())),
                            preferred_element_type=jnp.float32)
    prod = coeff * nf_ref[...]
    out_ref[...] = jnp.sum(prod.reshape(_P, _H, _C), axis=1)


def _conv(nf, npos, coord_slice, kpt, wb):
    return pl.pallas_call(
        _conv_body,
        grid=(_PSL // _P,),
        in_specs=[
            pl.BlockSpec((_E, _C), lambda i: (i, 0)),
            pl.BlockSpec((_E, 16), lambda i: (i, 0)),
            pl.BlockSpec((_P, 16), lambda i: (i, 0)),
            pl.BlockSpec((16, _K), lambda i: (0, 0)),
            pl.BlockSpec((_K, _C), lambda i: (0, 0)),
        ],
        out_specs=pl.BlockSpec((_P, _C), lambda i: (i, 0)),
        out_shape=jax.ShapeDtypeStruct((_PSL, _C), jnp.float32),
    )(nf, npos, coord_slice, kpt, wb)


def _tail_body(y0_ref, y1_ref, y2_ref, y3_ref, y4_ref, feat_ref, w_ref,
               g2_ref, b2_ref, g3_ref, b3_ref, out_ref):
    y0 = jnp.concatenate([y0_ref[...], y1_ref[...], y2_ref[...],
                          y3_ref[...], y4_ref[...]], axis=0)
    mean2 = jnp.mean(y0, axis=0, keepdims=True)
    var2 = jnp.mean(y0 * y0, axis=0, keepdims=True) - mean2 * mean2
    y = jnp.maximum((y0 - mean2) * lax.rsqrt(var2 + _EPS) * g2_ref[...]
                    + b2_ref[...], 0.0)
    w = w_ref[...]
    z = lax.dot_general(y, w, (((1,), (1,)), ((), ())),
                        preferred_element_type=jnp.float32)
    s3 = jnp.sum(y, axis=0, keepdims=True)
    mean3 = lax.dot_general(s3, w, (((1,), (1,)), ((), ()))) / _N
    gram = lax.dot_general(y, y, (((0,), (0,)), ((), ())),
                           preferred_element_type=jnp.float32)
    m = lax.dot_general(w, gram, (((1,), (0,)), ((), ())),
                        preferred_element_type=jnp.float32)
    q = (jnp.sum(m * w, axis=1) / _N)[None, :]
    var3 = q - mean3 * mean3
    scale = g3_ref[...] * lax.rsqrt(var3 + _EPS)
    shift = b3_ref[...] - mean3 * scale
    out_ref[...] = jnp.maximum(feat_ref[...] + z * scale + shift, 0.0)


def _tail(ys, feat, w3, g2, b2, g3, b3):
    return pl.pallas_call(
        _tail_body,
        out_shape=jax.ShapeDtypeStruct((_N, _C), jnp.float32),
    )(*ys, feat, w3, g2, b2, g3, b3)


def kernel(coord, feat, offset, reference_index, W_fc1, W_fc3,
           kernel_points, conv_weights,
           gamma1, beta1, gamma2, beta2, gamma3, beta3):
    coordpad = jnp.pad(coord, ((0, 0), (0, 13)))
    kpt = jnp.pad(kernel_points, ((0, 0), (0, 13))).T
    idx_flat = reference_index.reshape(-1)
    wb = conv_weights.astype(jnp.bfloat16)

    x1 = _fc1(feat, W_fc1, gamma1.reshape(1, -1), beta1.reshape(1, -1))
    ys = []
    for s in range(_NSL):
        nf_s, np_s = _sc_gather(x1, coordpad,
                                idx_flat[s * _ESL:(s + 1) * _ESL])
        ys.append(_conv(nf_s, np_s,
                        coordpad[s * _PSL:(s + 1) * _PSL], kpt, wb))
    x = _tail(ys, feat, W_fc3, gamma2.reshape(1, -1), beta2.reshape(1, -1),
              gamma3.reshape(1, -1), beta3.reshape(1, -1))
    return (coord, x, offset)


# speedup vs baseline: 1.1398x; 1.1398x over previous
import functools

import jax
import jax.numpy as jnp
from jax import lax
from jax.experimental import pallas as pl
from jax.experimental.pallas import tpu as pltpu
from jax.experimental.pallas import tpu_sc as plsc

_N = 10000
_C = 128
_H = 32
_K = 43
_EPS = 1e-5


def _fc1_body(feat_ref, w_ref, g_ref, b_ref, out_ref):
    f = feat_ref[...]
    w = w_ref[...]
    xw = lax.dot_general(f, w, (((1,), (1,)), ((), ())),
                         preferred_element_type=jnp.float32)
    s = jnp.sum(f, axis=0, keepdims=True)
    mean = lax.dot_general(s, w, (((1,), (1,)), ((), ()))) / _N
    gram = lax.dot_general(f, f, (((0,), (0,)), ((), ())),
                           preferred_element_type=jnp.float32)
    m = lax.dot_general(w, gram, (((1,), (0,)), ((), ())),
                        preferred_element_type=jnp.float32)
    q = (jnp.sum(m * w, axis=1) / _N)[None, :]
    var = q - mean * mean
    scale = g_ref[...] * lax.rsqrt(var + _EPS)
    shift = b_ref[...] - mean * scale
    out_ref[...] = jnp.maximum(xw * scale + shift, 0.0)


def _fc1(feat, w1, g1, b1):
    return pl.pallas_call(
        _fc1_body,
        out_shape=jax.ShapeDtypeStruct((_N, _C), jnp.float32),
    )(feat, w1, g1, b1)


_NW = 32
_ECH = 128
_NSL = 5
_PSL = _N // _NSL
_ESL = _PSL * _H
_NCHS = _ESL // _ECH
_NCHW = _NCHS // _NW
_LEFT = _NCHS - _NCHW * _NW


def _make_gather_body(width):
    def body(xt_ref, idx_ref, out_ref, idx0, idx1, b0, b1, sem0, sem1):
        cid = lax.axis_index("c")
        sid = lax.axis_index("s")
        wid = sid * 2 + cid

        def fire(j, idx_v, b_v, sem):
            base = (j * _NW + wid) * _ECH
            pltpu.sync_copy(idx_ref.at[pl.ds(base, _ECH)], idx_v)
            pltpu.async_copy(xt_ref.at[idx_v], b_v, sem)

        def drain(j, idx_v, b_v, sem):
            base = (j * _NW + wid) * _ECH
            pltpu.make_async_copy(xt_ref.at[idx_v], b_v, sem).wait()
            pltpu.sync_copy(b_v, out_ref.at[pl.ds(base, _ECH)])

        fire(0, idx0, b0, sem0)

        def step(jp, carry):
            j1 = 2 * jp + 1
            j2 = 2 * jp + 2

            @pl.when(j1 < _NCHW)
            def _():
                fire(j1, idx1, b1, sem1)

            drain(2 * jp, idx0, b0, sem0)

            @pl.when(j2 < _NCHW)
            def _():
                fire(j2, idx0, b0, sem0)

            @pl.when(j1 < _NCHW)
            def _():
                drain(j1, idx1, b1, sem1)

            return carry

        lax.fori_loop(0, (_NCHW + 1) // 2, step, 0)

        @pl.when(wid < _LEFT)
        def _():
            fire(_NCHW, idx0, b0, sem0)
            drain(_NCHW, idx0, b0, sem0)

    return body


def _sc_gather_feat(x1, idx_slice):
    k = pl.kernel(
        _make_gather_body(_C),
        out_type=jax.ShapeDtypeStruct((_ESL, _C), jnp.float32),
        mesh=plsc.VectorSubcoreMesh(core_axis_name="c", subcore_axis_name="s"),
        scratch_types=[pltpu.VMEM((_ECH,), jnp.int32),
                       pltpu.VMEM((_ECH,), jnp.int32),
                       pltpu.VMEM((_ECH, _C), jnp.float32),
                       pltpu.VMEM((_ECH, _C), jnp.float32),
                       pltpu.SemaphoreType.DMA,
                       pltpu.SemaphoreType.DMA],
    )
    return k(x1, idx_slice)


def _sc_gather_coord(coordpad, idx_slice):
    k = pl.kernel(
        _make_gather_body(16),
        out_type=jax.ShapeDtypeStruct((_ESL, 16), jnp.float32),
        mesh=plsc.VectorSubcoreMesh(core_axis_name="c", subcore_axis_name="s"),
        scratch_types=[pltpu.VMEM((_ECH,), jnp.int32),
                       pltpu.VMEM((_ECH,), jnp.int32),
                       pltpu.VMEM((_ECH, 16), jnp.float32),
                       pltpu.VMEM((_ECH, 16), jnp.float32),
                       pltpu.SemaphoreType.DMA,
                       pltpu.SemaphoreType.DMA],
        compiler_params=pltpu.CompilerParams(use_tc_tiling_on_sc=False),
    )
    return k(coordpad, idx_slice)


_P = 200
_E = _P * _H
_NB = _N // _P


def _conv_body(nf_ref, np_ref, cp_ref, kpt_ref, w_ref, out_ref):
    npos = np_ref[...]
    ctr = cp_ref[...]
    rel = (npos.reshape(_P, _H, 16) - ctr.reshape(_P, 1, 16)).reshape(_E, 16)
    kpt = kpt_ref[...]
    rn2 = jnp.sum(rel * rel, axis=1, keepdims=True)
    dotp = lax.dot_general(rel, kpt, (((1,), (0,)), ((), ())),
                           precision=lax.Precision.HIGHEST,
                           preferred_element_type=jnp.float32)
    kn2 = jnp.sum(kpt * kpt, axis=0, keepdims=True)
    sq = jnp.maximum(rn2 - 2.0 * dotp + kn2, 0.0)
    infl = jnp.maximum(1.0 - jnp.sqrt(sq), 0.0)
    coeff = lax.dot_general(infl.astype(jnp.bfloat16), w_ref[...],
                            (((1,), (0,)), ((), ())),
                            preferred_element_type=jnp.float32)
    prod = coeff * nf_ref[...]
    out_ref[...] = jnp.sum(prod.reshape(_P, _H, _C), axis=1)


def _conv(nf, npos, coord_slice, kpt, wb):
    return pl.pallas_call(
        _conv_body,
        grid=(_PSL // _P,),
        in_specs=[
            pl.BlockSpec((_E, _C), lambda i: (i, 0)),
            pl.BlockSpec((_E, 16), lambda i: (i, 0)),
            pl.BlockSpec((_P, 16), lambda i: (i, 0)),
            pl.BlockSpec((16, _K), lambda i: (0, 0)),
            pl.BlockSpec((_K, _C), lambda i: (0, 0)),
        ],
        out_specs=pl.BlockSpec((_P, _C), lambda i: (i, 0)),
        out_shape=jax.ShapeDtypeStruct((_PSL, _C), jnp.float32),
    )(nf, npos, coord_slice, kpt, wb)


def _tail_body(y0_ref, y1_ref, y2_ref, y3_ref, y4_ref, feat_ref, w_ref,
               g2_ref, b2_ref, g3_ref, b3_ref, out_ref):
    y0 = jnp.concatenate([y0_ref[...], y1_ref[...], y2_ref[...],
                          y3_ref[...], y4_ref[...]], axis=0)
    mean2 = jnp.mean(y0, axis=0, keepdims=True)
    var2 = jnp.mean(y0 * y0, axis=0, keepdims=True) - mean2 * mean2
    y = jnp.maximum((y0 - mean2) * lax.rsqrt(var2 + _EPS) * g2_ref[...]
                    + b2_ref[...], 0.0)
    w = w_ref[...]
    z = lax.dot_general(y, w, (((1,), (1,)), ((), ())),
                        preferred_element_type=jnp.float32)
    s3 = jnp.sum(y, axis=0, keepdims=True)
    mean3 = lax.dot_general(s3, w, (((1,), (1,)), ((), ()))) / _N
    gram = lax.dot_general(y, y, (((0,), (0,)), ((), ())),
                           preferred_element_type=jnp.float32)
    m = lax.dot_general(w, gram, (((1,), (0,)), ((), ())),
                        preferred_element_type=jnp.float32)
    q = (jnp.sum(m * w, axis=1) / _N)[None, :]
    var3 = q - mean3 * mean3
    scale = g3_ref[...] * lax.rsqrt(var3 + _EPS)
    shift = b3_ref[...] - mean3 * scale
    out_ref[...] = jnp.maximum(feat_ref[...] + z * scale + shift, 0.0)


def _tail(ys, feat, w3, g2, b2, g3, b3):
    return pl.pallas_call(
        _tail_body,
        out_shape=jax.ShapeDtypeStruct((_N, _C), jnp.float32),
    )(*ys, feat, w3, g2, b2, g3, b3)


def kernel(coord, feat, offset, reference_index, W_fc1, W_fc3,
           kernel_points, conv_weights,
           gamma1, beta1, gamma2, beta2, gamma3, beta3):
    coordpad = jnp.pad(coord, ((0, 0), (0, 13)))
    kpt = jnp.pad(kernel_points, ((0, 0), (0, 13))).T
    idx_flat = reference_index.reshape(-1)
    wb = conv_weights.astype(jnp.bfloat16)

    x1 = _fc1(feat, W_fc1, gamma1.reshape(1, -1), beta1.reshape(1, -1))
    ys = []
    for s in range(_NSL):
        idx_s = idx_flat[s * _ESL:(s + 1) * _ESL]
        nf_s = _sc_gather_feat(x1, idx_s)
        np_s = _sc_gather_coord(coordpad, idx_s)
        ys.append(_conv(nf_s, np_s,
                        coordpad[s * _PSL:(s + 1) * _PSL], kpt, wb))
    x = _tail(ys, feat, W_fc3, gamma2.reshape(1, -1), beta2.reshape(1, -1),
              gamma3.reshape(1, -1), beta3.reshape(1, -1))
    return (coord, x, offset)


# speedup vs baseline: 1.2108x; 1.0623x over previous
import functools

import jax
import jax.numpy as jnp
from jax import lax
from jax.experimental import pallas as pl
from jax.experimental.pallas import tpu as pltpu
from jax.experimental.pallas import tpu_sc as plsc

_N = 10000
_C = 128
_H = 32
_K = 43
_EPS = 1e-5


def _fc1_body(feat_ref, w_ref, g_ref, b_ref, out_ref):
    f = feat_ref[...]
    w = w_ref[...]
    xw = lax.dot_general(f, w, (((1,), (1,)), ((), ())),
                         preferred_element_type=jnp.float32)
    s = jnp.sum(f, axis=0, keepdims=True)
    mean = lax.dot_general(s, w, (((1,), (1,)), ((), ()))) / _N
    gram = lax.dot_general(f, f, (((0,), (0,)), ((), ())),
                           preferred_element_type=jnp.float32)
    m = lax.dot_general(w, gram, (((1,), (0,)), ((), ())),
                        preferred_element_type=jnp.float32)
    q = (jnp.sum(m * w, axis=1) / _N)[None, :]
    var = q - mean * mean
    scale = g_ref[...] * lax.rsqrt(var + _EPS)
    shift = b_ref[...] - mean * scale
    out_ref[...] = jnp.maximum(xw * scale + shift, 0.0)


def _fc1(feat, w1, g1, b1):
    return pl.pallas_call(
        _fc1_body,
        out_shape=jax.ShapeDtypeStruct((_N, _C), jnp.float32),
    )(feat, w1, g1, b1)


_NW = 32
_ECH = 128
_NSL = 5
_PSL = _N // _NSL
_ESL = _PSL * _H
_NCHS = _ESL // _ECH
_NCHW = _NCHS // _NW
_LEFT = _NCHS - _NCHW * _NW


def _sc_gather_body(xt_ref, ct_ref, idx_ref, nf_ref, np_ref,
                    idx0, idx1, rows0, rows1, crows0, crows1,
                    semf0, semf1, semc0, semc1):
    cid = lax.axis_index("c")
    sid = lax.axis_index("s")
    wid = sid * 2 + cid

    def fire(j, idx_v, rows_v, crows_v, semf, semc):
        base = (j * _NW + wid) * _ECH
        pltpu.sync_copy(idx_ref.at[pl.ds(base, _ECH)], idx_v)
        pltpu.async_copy(xt_ref.at[idx_v], rows_v, semf)
        pltpu.async_copy(ct_ref.at[idx_v], crows_v, semc)

    def drain(j, idx_v, rows_v, crows_v, semf, semc):
        base = (j * _NW + wid) * _ECH
        pltpu.make_async_copy(xt_ref.at[idx_v], rows_v, semf).wait()
        pltpu.make_async_copy(ct_ref.at[idx_v], crows_v, semc).wait()
        pltpu.sync_copy(rows_v, nf_ref.at[pl.ds(base, _ECH)])
        pltpu.sync_copy(crows_v, np_ref.at[pl.ds(base, _ECH)])

    fire(0, idx0, rows0, crows0, semf0, semc0)

    def step(jp, carry):
        j1 = 2 * jp + 1
        j2 = 2 * jp + 2

        @pl.when(j1 < _NCHW)
        def _():
            fire(j1, idx1, rows1, crows1, semf1, semc1)

        drain(2 * jp, idx0, rows0, crows0, semf0, semc0)

        @pl.when(j2 < _NCHW)
        def _():
            fire(j2, idx0, rows0, crows0, semf0, semc0)

        @pl.when(j1 < _NCHW)
        def _():
            drain(j1, idx1, rows1, crows1, semf1, semc1)

        return carry

    lax.fori_loop(0, (_NCHW + 1) // 2, step, 0)

    @pl.when(wid < _LEFT)
    def _():
        fire(_NCHW, idx0, rows0, crows0, semf0, semc0)
        drain(_NCHW, idx0, rows0, crows0, semf0, semc0)


def _sc_gather_both(x1, coordpad128, idx_slice):
    k = pl.kernel(
        _sc_gather_body,
        out_type=[jax.ShapeDtypeStruct((_ESL, _C), jnp.float32),
                  jax.ShapeDtypeStruct((_ESL, 128), jnp.float32)],
        mesh=plsc.VectorSubcoreMesh(core_axis_name="c", subcore_axis_name="s"),
        scratch_types=[pltpu.VMEM((_ECH,), jnp.int32),
                       pltpu.VMEM((_ECH,), jnp.int32),
                       pltpu.VMEM((_ECH, _C), jnp.float32),
                       pltpu.VMEM((_ECH, _C), jnp.float32),
                       pltpu.VMEM((_ECH, 128), jnp.float32),
                       pltpu.VMEM((_ECH, 128), jnp.float32),
                       pltpu.SemaphoreType.DMA,
                       pltpu.SemaphoreType.DMA,
                       pltpu.SemaphoreType.DMA,
                       pltpu.SemaphoreType.DMA],
    )
    return k(x1, coordpad128, idx_slice)


_P = 200
_E = _P * _H
_NB = _N // _P


def _conv_body(nf_ref, np_ref, cp_ref, kpt_ref, w_ref, out_ref):
    npos = np_ref[...][:, 0:16]
    ctr = cp_ref[...]
    rel = (npos.reshape(_P, _H, 16) - ctr.reshape(_P, 1, 16)).reshape(_E, 16)
    kpt = kpt_ref[...]
    rn2 = jnp.sum(rel * rel, axis=1, keepdims=True)
    dotp = lax.dot_general(rel, kpt, (((1,), (0,)), ((), ())),
                           precision=lax.Precision.HIGHEST,
                           preferred_element_type=jnp.float32)
    kn2 = jnp.sum(kpt * kpt, axis=0, keepdims=True)
    sq = jnp.maximum(rn2 - 2.0 * dotp + kn2, 0.0)
    infl = jnp.maximum(1.0 - jnp.sqrt(sq), 0.0)
    coeff = lax.dot_general(infl.astype(jnp.bfloat16), w_ref[...],
                            (((1,), (0,)), ((), ())),
                            preferred_element_type=jnp.float32)
    prod = coeff * nf_ref[...]
    out_ref[...] = jnp.sum(prod.reshape(_P, _H, _C), axis=1)


def _conv(nf, npos, coord_slice, kpt, wb):
    return pl.pallas_call(
        _conv_body,
        grid=(_PSL // _P,),
        in_specs=[
            pl.BlockSpec((_E, _C), lambda i: (i, 0)),
            pl.BlockSpec((_E, 128), lambda i: (i, 0)),
            pl.BlockSpec((_P, 16), lambda i: (i, 0)),
            pl.BlockSpec((16, _K), lambda i: (0, 0)),
            pl.BlockSpec((_K, _C), lambda i: (0, 0)),
        ],
        out_specs=pl.BlockSpec((_P, _C), lambda i: (i, 0)),
        out_shape=jax.ShapeDtypeStruct((_PSL, _C), jnp.float32),
    )(nf, npos, coord_slice, kpt, wb)


def _tail_body(y0_ref, y1_ref, y2_ref, y3_ref, y4_ref, feat_ref, w_ref,
               g2_ref, b2_ref, g3_ref, b3_ref, out_ref):
    y0 = jnp.concatenate([y0_ref[...], y1_ref[...], y2_ref[...],
                          y3_ref[...], y4_ref[...]], axis=0)
    mean2 = jnp.mean(y0, axis=0, keepdims=True)
    var2 = jnp.mean(y0 * y0, axis=0, keepdims=True) - mean2 * mean2
    y = jnp.maximum((y0 - mean2) * lax.rsqrt(var2 + _EPS) * g2_ref[...]
                    + b2_ref[...], 0.0)
    w = w_ref[...]
    z = lax.dot_general(y, w, (((1,), (1,)), ((), ())),
                        preferred_element_type=jnp.float32)
    s3 = jnp.sum(y, axis=0, keepdims=True)
    mean3 = lax.dot_general(s3, w, (((1,), (1,)), ((), ()))) / _N
    gram = lax.dot_general(y, y, (((0,), (0,)), ((), ())),
                           preferred_element_type=jnp.float32)
    m = lax.dot_general(w, gram, (((1,), (0,)), ((), ())),
                        preferred_element_type=jnp.float32)
    q = (jnp.sum(m * w, axis=1) / _N)[None, :]
    var3 = q - mean3 * mean3
    scale = g3_ref[...] * lax.rsqrt(var3 + _EPS)
    shift = b3_ref[...] - mean3 * scale
    out_ref[...] = jnp.maximum(feat_ref[...] + z * scale + shift, 0.0)


def _tail(ys, feat, w3, g2, b2, g3, b3):
    return pl.pallas_call(
        _tail_body,
        out_shape=jax.ShapeDtypeStruct((_N, _C), jnp.float32),
    )(*ys, feat, w3, g2, b2, g3, b3)


def kernel(coord, feat, offset, reference_index, W_fc1, W_fc3,
           kernel_points, conv_weights,
           gamma1, beta1, gamma2, beta2, gamma3, beta3):
    coordpad = jnp.pad(coord, ((0, 0), (0, 13)))
    coordpad128 = jnp.pad(coord, ((0, 0), (0, 125)))
    kpt = jnp.pad(kernel_points, ((0, 0), (0, 13))).T
    idx_flat = reference_index.reshape(-1)
    wb = conv_weights.astype(jnp.bfloat16)

    x1 = _fc1(feat, W_fc1, gamma1.reshape(1, -1), beta1.reshape(1, -1))
    ys = []
    for s in range(_NSL):
        idx_s = idx_flat[s * _ESL:(s + 1) * _ESL]
        nf_s, np_s = _sc_gather_both(x1, coordpad128, idx_s)
        ys.append(_conv(nf_s, np_s,
                        coordpad[s * _PSL:(s + 1) * _PSL], kpt, wb))
    x = _tail(ys, feat, W_fc3, gamma2.reshape(1, -1), beta2.reshape(1, -1),
              gamma3.reshape(1, -1), beta3.reshape(1, -1))
    return (coord, x, offset)
